# baseline (speedup 1.0000x reference)
import functools
import math

import jax
import jax.numpy as jnp
from jax import lax
from jax.experimental import pallas as pl
from jax.experimental.pallas import tpu as pltpu
from jax.experimental.pallas import tpu_sc as plsc

NPRED = 1_000_000
BATCH = 16384
DUR_N = 11
TOTAL = float(DUR_N - 1)

_PSTR = 7936
_PROWS = _PSTR * 128
_VR = 2 * _PSTR
_PW = _PSTR * 256

_NC, _NS = 2, 16
_NW = _NC * _NS
_BPW = BATCH // _NW

_EPS = float(jnp.finfo(jnp.float32).eps)
_LOGC = [
    math.lgamma(DUR_N) - math.lgamma(j + 1.0) - math.lgamma(TOTAL - j + 1.0)
    for j in range(DUR_N)
]


def _gather_sc(view1d, idx1, idx2):
    mesh = plsc.VectorSubcoreMesh(core_axis_name="c", subcore_axis_name="s")

    @functools.partial(
        pl.kernel,
        mesh=mesh,
        out_type=jax.ShapeDtypeStruct((4, 128, 128), jnp.float32),
        scratch_types=[
            pltpu.VMEM((_BPW,), jnp.int32),
            pltpu.VMEM((4, 4, 128), jnp.int32),
            pltpu.VMEM((16, 128), jnp.float32),
            pltpu.SemaphoreType.DMA,
        ],
    )
    def kg(tab, i1, i2, g_out, raw_v, adr_v, rows_v, sem):
        wid = lax.axis_index("s") * _NC + lax.axis_index("c")
        for t, src in enumerate((i1, i2)):
            pltpu.sync_copy(src.at[pl.ds(wid * _BPW, _BPW)], raw_v)
            for i in range(_BPW // 16):
                v = raw_v[pl.ds(i * 16, 16)]
                a0 = (v >> 7) * 256 + (v & 127)
                rr, cc = i // 8, (i % 8) * 16
                adr_v[2 * t, rr, pl.ds(cc, 16)] = a0
                adr_v[2 * t + 1, rr, pl.ds(cc, 16)] = a0 + 128
        gd = [
            pltpu.async_copy(tab.at[adr_v.at[r, j]], rows_v.at[4 * r + j],
                             sem)
            for r in range(4) for j in range(4)
        ]
        for d in gd:
            d.wait()
        for r in range(4):
            pltpu.sync_copy(rows_v.at[pl.ds(4 * r, 4), :],
                            g_out.at[r, pl.ds(wid * 4, 4), :])

    return kg(view1d, idx1, idx2)


def _small_body(m_ref, k_ref, g_ref,
                b1_ref, d1_ref, b2_ref, d2_ref, q1_ref, q2_ref):
    minv = m_ref[0, 0]
    kk = k_ref[0, 0]
    for t, (b_ref, d_ref, q_ref) in enumerate(
            ((b1_ref, d1_ref, q1_ref), (b2_ref, d2_ref, q2_ref))):
        a = g_ref[2 * t]
        d = g_ref[2 * t + 1]
        dur = d * d
        b_ref[...] = a * a - minv
        d_ref[...] = dur
        x = kk * jnp.log(dur)
        p = jax.nn.sigmoid(x)
        p = jnp.clip(p, _EPS, 1.0 - _EPS)
        logits = jnp.log(p) - jnp.log1p(-p)
        neg_max = jnp.minimum(logits, 0.0)
        base = TOTAL * neg_max - TOTAL * jnp.log(
            jnp.exp(neg_max) + jnp.exp(-logits + neg_max))
        for j in range(DUR_N):
            q_ref[j // 8, :, j % 8, :] = _LOGC[j] + float(j) * logits + base
        for j in range(DUR_N, 16):
            q_ref[j // 8, :, j % 8, :] = jnp.zeros_like(base)


def kernel(idx1, idx2, pred_tensor, k):
    padded = jnp.pad(pred_tensor, ((0, _PROWS - NPRED), (0, 0)),
                     constant_values=jnp.inf)
    viewp = (padded.reshape(_PSTR, 128, 2)
             .transpose(0, 2, 1)
             .reshape(_VR, 128))
    view1d = viewp.reshape(_PW)

    g = _gather_sc(view1d, idx1, idx2)

    def _minanch_one(x_ref, o_ref, mo_ref):
        x = x_ref[...]
        sq = x * x
        row = lax.broadcasted_iota(jnp.int32, x.shape, 0)
        m = jnp.min(jnp.where(row % 2 == 0, sq, jnp.inf))
        mo_ref[0, 0] = m
        o_ref[...] = jnp.where(row % 2 == 0, sq - m, sq)

    anch_v, minv = pl.pallas_call(
        _minanch_one,
        in_specs=[pl.BlockSpec((_VR, 128), lambda: (0, 0))],
        out_specs=[
            pl.BlockSpec((_VR, 128), lambda: (0, 0)),
            pl.BlockSpec(memory_space=pltpu.SMEM),
        ],
        out_shape=[
            jax.ShapeDtypeStruct((_VR, 128), jnp.float32),
            jax.ShapeDtypeStruct((1, 1), jnp.float32),
        ],
    )(viewp)

    anchored = (anch_v.reshape(_PSTR, 2, 128)
                .transpose(0, 2, 1)
                .reshape(_PROWS, 2)[:NPRED])

    k2 = k.reshape(1, 1)
    b1, d1, b2, d2, q1, q2 = pl.pallas_call(
        _small_body,
        in_specs=[
            pl.BlockSpec(memory_space=pltpu.SMEM),
            pl.BlockSpec(memory_space=pltpu.SMEM),
            pl.BlockSpec((4, 128, 128), lambda: (0, 0, 0)),
        ],
        out_specs=[
            pl.BlockSpec((128, 128), lambda: (0, 0)),
            pl.BlockSpec((128, 128), lambda: (0, 0)),
            pl.BlockSpec((128, 128), lambda: (0, 0)),
            pl.BlockSpec((128, 128), lambda: (0, 0)),
            pl.BlockSpec((2, 128, 8, 128), lambda: (0, 0, 0, 0)),
            pl.BlockSpec((2, 128, 8, 128), lambda: (0, 0, 0, 0)),
        ],
        out_shape=[
            jax.ShapeDtypeStruct((128, 128), jnp.float32),
            jax.ShapeDtypeStruct((128, 128), jnp.float32),
            jax.ShapeDtypeStruct((128, 128), jnp.float32),
            jax.ShapeDtypeStruct((128, 128), jnp.float32),
            jax.ShapeDtypeStruct((2, 128, 8, 128), jnp.float32),
            jax.ShapeDtypeStruct((2, 128, 8, 128), jnp.float32),
        ],
    )(minv, k2, g)

    p1 = (q1.transpose(0, 2, 1, 3).reshape(16, BATCH).T)[:, :DUR_N]
    p2 = (q2.transpose(0, 2, 1, 3).reshape(16, BATCH).T)[:, :DUR_N]

    return (
        b1.reshape(BATCH),
        d1.reshape(BATCH),
        b2.reshape(BATCH),
        d2.reshape(BATCH),
        p1,
        p2,
        anchored,
    )
